# baseline (speedup 1.0000x reference)
import functools

import numpy as np
import jax
import jax.numpy as jnp
from jax import lax
from jax.experimental import pallas as pl
from jax.experimental.pallas import tpu as pltpu
from jax.experimental.pallas import tpu_sc as plsc

_B = 131072
_L = 16
_T = 14
_F = 2
_TBL = (1 << _T) * _F
_MASK = np.int32((1 << _T) - 1)
_P2 = np.int32(np.int64(2654435761) - (np.int64(1) << np.int64(32)))

_NC = 2
_NS = 16
_LANES = 16
_HALF = _B // _NC
_CHUNK = 8192
_NCHUNK = _HALF // _CHUNK
_NVEC = _CHUNK // _LANES

_CB = 8192
_Z = np.int32(0)


def _resolutions_f32() -> np.ndarray:
    b_geo = np.exp((np.log(512.0) - np.log(16.0)) / (_L - 1))
    return np.array([np.floor(16.0 * b_geo ** i) for i in range(_L)],
                    dtype=np.float32)


def _encode_body(xt_hbm, tab_hbm, res_hbm, out_hbm,
                 tab0_v, tab1_v, res_v, xin_a, xin_b, o0_a, o1_a, o0_b, o1_b,
                 sem_tab, sem_in, sem_out):
    c = lax.axis_index("c")
    s = lax.axis_index("s")
    base = c * np.int32(_HALF)
    orow0 = s * np.int32(2 * _B)
    orow1 = orow0 + np.int32(_B)

    def in_off(k):
        off = pl.multiple_of(base + np.int32(k * _CHUNK), _CHUNK)
        return pl.multiple_of(off + off, 2 * _CHUNK)

    xbufs = [xin_a, xin_b]
    obufs = [(o0_a, o1_a), (o0_b, o1_b)]
    in_cp = [None] * _NCHUNK
    out_cp = [None] * _NCHUNK
    in_cp[0] = pltpu.async_copy(
        xt_hbm.at[pl.ds(in_off(0), 2 * _CHUNK)], xin_a, sem_in)
    half_tbl = np.int32(_TBL // 2)
    tb = pl.multiple_of(s * np.int32(_TBL), _TBL)
    tab_cp0 = pltpu.async_copy(tab_hbm.at[pl.ds(tb, _TBL // 2)], tab0_v, sem_tab)
    tab_cp1 = pltpu.async_copy(
        tab_hbm.at[pl.ds(pl.multiple_of(tb + half_tbl, _TBL // 2), _TBL // 2)],
        tab1_v, sem_tab)
    pltpu.sync_copy(res_hbm, res_v)
    resv = plsc.load_gather(res_v, [jnp.full((_LANES,), s, dtype=jnp.int32)])
    tab_cp0.wait()
    tab_cp1.wait()

    for k in range(_NCHUNK):
        off = pl.multiple_of(base + np.int32(k * _CHUNK), _CHUNK)
        xin_v = xbufs[k % 2]
        o0_v, o1_v = obufs[k % 2]
        if k + 1 < _NCHUNK:
            in_cp[k + 1] = pltpu.async_copy(
                xt_hbm.at[pl.ds(in_off(k + 1), 2 * _CHUNK)],
                xbufs[(k + 1) % 2], sem_in)
        in_cp[k].wait()
        if k >= 2:
            out_cp[k - 2][0].wait()
            out_cp[k - 2][1].wait()

        @plsc.parallel_loop(np.int32(0), np.int32(_CHUNK),
                            step=np.int32(_LANES), unroll=8)
        def blk_body(st):
            xo = ((st >> np.int32(7)) << np.int32(8)) + (st & np.int32(127))
            if True:
                so = np.int32(0)
                pt = st
                x0 = xin_v[pl.ds(xo, _LANES)]
                x1 = xin_v[pl.ds(xo + np.int32(128), _LANES)]
                fx0 = x0 * resv
                fx1 = x1 * resv
                i0 = fx0.astype(jnp.int32)
                i1 = fx1.astype(jnp.int32)
                f0 = fx0 - i0.astype(jnp.float32)
                f1 = fx1 - i1.astype(jnp.float32)
                one = np.int32(1)
                ha = i1 * _P2
                hb = ha + _P2
                i0p = i0 + one
                e00 = (i0 ^ ha) & _MASK
                e10 = (i0p ^ ha) & _MASK
                e01 = (i0 ^ hb) & _MASK
                e11 = (i0p ^ hb) & _MASK
                v00a = plsc.load_gather(tab0_v, [e00])
                v00b = plsc.load_gather(tab1_v, [e00])
                v10a = plsc.load_gather(tab0_v, [e10])
                v10b = plsc.load_gather(tab1_v, [e10])
                v01a = plsc.load_gather(tab0_v, [e01])
                v01b = plsc.load_gather(tab1_v, [e01])
                v11a = plsc.load_gather(tab0_v, [e11])
                v11b = plsc.load_gather(tab1_v, [e11])
                a0 = v00a + f0 * (v10a - v00a)
                a1 = v01a + f0 * (v11a - v01a)
                b0 = v00b + f0 * (v10b - v00b)
                b1 = v01b + f0 * (v11b - v01b)
                o0_v[pl.ds(pt + so, _LANES)] = a0 + f1 * (a1 - a0)
                o1_v[pl.ds(pt + so, _LANES)] = b0 + f1 * (b1 - b0)

        out_cp[k] = (
            pltpu.async_copy(
                o0_v,
                out_hbm.at[pl.ds(pl.multiple_of(orow0 + off, _CHUNK), _CHUNK)],
                sem_out),
            pltpu.async_copy(
                o1_v,
                out_hbm.at[pl.ds(pl.multiple_of(orow1 + off, _CHUNK), _CHUNK)],
                sem_out),
        )

    for k in range(max(0, _NCHUNK - 2), _NCHUNK):
        out_cp[k][0].wait()
        out_cp[k][1].wait()


def _sc_encode(xt, tabf, resf):
    mesh = plsc.VectorSubcoreMesh(core_axis_name="c", subcore_axis_name="s",
                                  num_cores=_NC, num_subcores=_NS)
    return pl.kernel(
        _encode_body,
        out_type=jax.ShapeDtypeStruct((_L * _F * _B,), jnp.float32),
        mesh=mesh,
        compiler_params=pltpu.CompilerParams(needs_layout_passes=False),
        scratch_types=[
            pltpu.VMEM((_TBL // 2,), jnp.float32),
            pltpu.VMEM((_TBL // 2,), jnp.float32),
            pltpu.VMEM((_LANES,), jnp.float32),
            pltpu.VMEM((2 * _CHUNK,), jnp.float32),
            pltpu.VMEM((2 * _CHUNK,), jnp.float32),
            pltpu.VMEM((_CHUNK,), jnp.float32),
            pltpu.VMEM((_CHUNK,), jnp.float32),
            pltpu.VMEM((_CHUNK,), jnp.float32),
            pltpu.VMEM((_CHUNK,), jnp.float32),
            pltpu.SemaphoreType.DMA,
            pltpu.SemaphoreType.DMA,
            pltpu.SemaphoreType.DMA,
        ],
    )(xt, tabf, resf)


def _mlp_body(f_ref, w1_ref, b1_ref, w2_ref, b2_ref, w3_ref, b3_ref, o_ref):
    fb = f_ref[...]
    h1 = lax.dot_general(w1_ref[...], fb, (((1,), (0,)), ((), ())),
                         preferred_element_type=jnp.float32) + b1_ref[...]
    h1 = jnp.maximum(h1, 0.0)
    h2 = lax.dot_general(w2_ref[...], h1, (((1,), (0,)), ((), ())),
                         preferred_element_type=jnp.float32) + b2_ref[...]
    h2 = jnp.maximum(h2, 0.0)
    o_ref[...] = lax.dot_general(w3_ref[...], h2, (((1,), (0,)), ((), ())),
                                 preferred_element_type=jnp.float32) + b3_ref[...]


def _tc_mlp(feats, w1, b1, w2, b2, w3, b3):
    grid = _B // _CB
    return pl.pallas_call(
        _mlp_body,
        grid=(grid,),
        in_specs=[
            pl.BlockSpec((_L * _F, _CB), lambda j: (_Z, j)),
            pl.BlockSpec((64, _L * _F), lambda j: (_Z, _Z)),
            pl.BlockSpec((64, 1), lambda j: (_Z, _Z)),
            pl.BlockSpec((64, 64), lambda j: (_Z, _Z)),
            pl.BlockSpec((64, 1), lambda j: (_Z, _Z)),
            pl.BlockSpec((3, 64), lambda j: (_Z, _Z)),
            pl.BlockSpec((3, 1), lambda j: (_Z, _Z)),
        ],
        out_specs=pl.BlockSpec((3, _CB), lambda j: (_Z, j)),
        out_shape=jax.ShapeDtypeStruct((3, _B), jnp.float32),
    )(feats, w1, b1, w2, b2, w3, b3)


def kernel(x, tables, W1, b1, W2, b2, W3, b3):
    xt = jnp.transpose(x.astype(jnp.float32).reshape(_B // 128, 128, 2),
                       (0, 2, 1)).reshape(-1)
    tabf = jnp.transpose(tables.astype(jnp.float32),
                         (0, 2, 1)).reshape(-1)
    resf = jnp.asarray(_resolutions_f32())
    feats = _sc_encode(xt, tabf, resf).reshape(_L * _F, _B)
    out3b = _tc_mlp(
        feats,
        W1.astype(jnp.float32), b1.astype(jnp.float32).reshape(64, 1),
        W2.astype(jnp.float32), b2.astype(jnp.float32).reshape(64, 1),
        W3.astype(jnp.float32), b3.astype(jnp.float32).reshape(3, 1),
    )
    return out3b.T.astype(W1.dtype)


# speedup vs baseline: 1.2166x; 1.2166x over previous
import functools

import numpy as np
import jax
import jax.numpy as jnp
from jax import lax
from jax.experimental import pallas as pl
from jax.experimental.pallas import tpu as pltpu
from jax.experimental.pallas import tpu_sc as plsc

_B = 131072
_L = 16
_T = 14
_F = 2
_TBL = (1 << _T) * _F
_MASK = np.int32((1 << _T) - 1)
_P2 = np.int32(np.int64(2654435761) - (np.int64(1) << np.int64(32)))

_NC = 2
_NS = 16
_LANES = 16
_HALF = _B // _NC
_CHUNK = 8192
_NCHUNK = _HALF // _CHUNK
_NVEC = _CHUNK // _LANES

_CB = 16384
_Z = np.int32(0)


def _resolutions_f32() -> np.ndarray:
    b_geo = np.exp((np.log(512.0) - np.log(16.0)) / (_L - 1))
    return np.array([np.floor(16.0 * b_geo ** i) for i in range(_L)],
                    dtype=np.float32)


def _encode_body(xt_hbm, tab_hbm, res_hbm, out_hbm,
                 tab0_v, tab1_v, res_v, xin_a, xin_b, o0_a, o1_a, o0_b, o1_b,
                 sem_tab, sem_in, sem_out):
    c = lax.axis_index("c")
    s = lax.axis_index("s")
    base = c * np.int32(_HALF)
    orow0 = s * np.int32(2 * _B)
    orow1 = orow0 + np.int32(_B)

    def in_off(k):
        off = pl.multiple_of(base + np.int32(k * _CHUNK), _CHUNK)
        return pl.multiple_of(off + off, 2 * _CHUNK)

    xbufs = [xin_a, xin_b]
    obufs = [(o0_a, o1_a), (o0_b, o1_b)]
    in_cp = [None] * _NCHUNK
    out_cp = [None] * _NCHUNK
    in_cp[0] = pltpu.async_copy(
        xt_hbm.at[pl.ds(in_off(0), 2 * _CHUNK)], xin_a, sem_in)
    half_tbl = np.int32(_TBL // 2)
    tb = pl.multiple_of(s * np.int32(_TBL), _TBL)
    tab_cp0 = pltpu.async_copy(tab_hbm.at[pl.ds(tb, _TBL // 2)], tab0_v, sem_tab)
    tab_cp1 = pltpu.async_copy(
        tab_hbm.at[pl.ds(pl.multiple_of(tb + half_tbl, _TBL // 2), _TBL // 2)],
        tab1_v, sem_tab)
    pltpu.sync_copy(res_hbm, res_v)
    resv = plsc.load_gather(res_v, [jnp.full((_LANES,), s, dtype=jnp.int32)])
    tab_cp0.wait()
    tab_cp1.wait()

    for k in range(_NCHUNK):
        off = pl.multiple_of(base + np.int32(k * _CHUNK), _CHUNK)
        xin_v = xbufs[k % 2]
        o0_v, o1_v = obufs[k % 2]
        if k + 1 < _NCHUNK:
            in_cp[k + 1] = pltpu.async_copy(
                xt_hbm.at[pl.ds(in_off(k + 1), 2 * _CHUNK)],
                xbufs[(k + 1) % 2], sem_in)
        in_cp[k].wait()
        if k >= 2:
            out_cp[k - 2][0].wait()
            out_cp[k - 2][1].wait()

        @plsc.parallel_loop(np.int32(0), np.int32(_CHUNK),
                            step=np.int32(_LANES), unroll=4)
        def blk_body(st):
            xo = ((st >> np.int32(7)) << np.int32(8)) + (st & np.int32(127))
            if True:
                so = np.int32(0)
                pt = st
                x0 = xin_v[pl.ds(xo, _LANES)]
                x1 = xin_v[pl.ds(xo + np.int32(128), _LANES)]
                fx0 = x0 * resv
                fx1 = x1 * resv
                i0 = fx0.astype(jnp.int32)
                i1 = fx1.astype(jnp.int32)
                f0 = fx0 - i0.astype(jnp.float32)
                f1 = fx1 - i1.astype(jnp.float32)
                one = np.int32(1)
                ha = i1 * _P2
                hb = ha + _P2
                i0p = i0 + one
                e00 = (i0 ^ ha) & _MASK
                e10 = (i0p ^ ha) & _MASK
                e01 = (i0 ^ hb) & _MASK
                e11 = (i0p ^ hb) & _MASK
                v00a = plsc.load_gather(tab0_v, [e00])
                v00b = plsc.load_gather(tab1_v, [e00])
                v10a = plsc.load_gather(tab0_v, [e10])
                v10b = plsc.load_gather(tab1_v, [e10])
                v01a = plsc.load_gather(tab0_v, [e01])
                v01b = plsc.load_gather(tab1_v, [e01])
                v11a = plsc.load_gather(tab0_v, [e11])
                v11b = plsc.load_gather(tab1_v, [e11])
                a0 = v00a + f0 * (v10a - v00a)
                a1 = v01a + f0 * (v11a - v01a)
                b0 = v00b + f0 * (v10b - v00b)
                b1 = v01b + f0 * (v11b - v01b)
                o0_v[pl.ds(pt + so, _LANES)] = a0 + f1 * (a1 - a0)
                o1_v[pl.ds(pt + so, _LANES)] = b0 + f1 * (b1 - b0)

        out_cp[k] = (
            pltpu.async_copy(
                o0_v,
                out_hbm.at[pl.ds(pl.multiple_of(orow0 + off, _CHUNK), _CHUNK)],
                sem_out),
            pltpu.async_copy(
                o1_v,
                out_hbm.at[pl.ds(pl.multiple_of(orow1 + off, _CHUNK), _CHUNK)],
                sem_out),
        )

    for k in range(max(0, _NCHUNK - 2), _NCHUNK):
        out_cp[k][0].wait()
        out_cp[k][1].wait()


def _sc_encode(xt, tabf, resf):
    mesh = plsc.VectorSubcoreMesh(core_axis_name="c", subcore_axis_name="s",
                                  num_cores=_NC, num_subcores=_NS)
    return pl.kernel(
        _encode_body,
        out_type=jax.ShapeDtypeStruct((_L * _F * _B,), jnp.float32),
        mesh=mesh,
        compiler_params=pltpu.CompilerParams(needs_layout_passes=False),
        scratch_types=[
            pltpu.VMEM((_TBL // 2,), jnp.float32),
            pltpu.VMEM((_TBL // 2,), jnp.float32),
            pltpu.VMEM((_LANES,), jnp.float32),
            pltpu.VMEM((2 * _CHUNK,), jnp.float32),
            pltpu.VMEM((2 * _CHUNK,), jnp.float32),
            pltpu.VMEM((_CHUNK,), jnp.float32),
            pltpu.VMEM((_CHUNK,), jnp.float32),
            pltpu.VMEM((_CHUNK,), jnp.float32),
            pltpu.VMEM((_CHUNK,), jnp.float32),
            pltpu.SemaphoreType.DMA,
            pltpu.SemaphoreType.DMA,
            pltpu.SemaphoreType.DMA,
        ],
    )(xt, tabf, resf)


def _mlp_body(f_ref, w1_ref, b1_ref, w2_ref, b2_ref, w3_ref, b3_ref, o_ref):
    fb = f_ref[...]
    h1 = lax.dot_general(w1_ref[...], fb, (((1,), (0,)), ((), ())),
                         preferred_element_type=jnp.float32) + b1_ref[...]
    h1 = jnp.maximum(h1, 0.0)
    h2 = lax.dot_general(w2_ref[...], h1, (((1,), (0,)), ((), ())),
                         preferred_element_type=jnp.float32) + b2_ref[...]
    h2 = jnp.maximum(h2, 0.0)
    o_ref[...] = lax.dot_general(w3_ref[...], h2, (((1,), (0,)), ((), ())),
                                 preferred_element_type=jnp.float32) + b3_ref[...]


def _tc_mlp(feats, w1, b1, w2, b2, w3, b3):
    grid = _B // _CB
    return pl.pallas_call(
        _mlp_body,
        grid=(grid,),
        in_specs=[
            pl.BlockSpec((_L * _F, _CB), lambda j: (_Z, j)),
            pl.BlockSpec((64, _L * _F), lambda j: (_Z, _Z)),
            pl.BlockSpec((64, 1), lambda j: (_Z, _Z)),
            pl.BlockSpec((64, 64), lambda j: (_Z, _Z)),
            pl.BlockSpec((64, 1), lambda j: (_Z, _Z)),
            pl.BlockSpec((3, 64), lambda j: (_Z, _Z)),
            pl.BlockSpec((3, 1), lambda j: (_Z, _Z)),
        ],
        out_specs=pl.BlockSpec((3, _CB), lambda j: (_Z, j)),
        out_shape=jax.ShapeDtypeStruct((3, _B), jnp.float32),
    )(feats, w1, b1, w2, b2, w3, b3)


def kernel(x, tables, W1, b1, W2, b2, W3, b3):
    xt = jnp.transpose(x.astype(jnp.float32).reshape(_B // 128, 128, 2),
                       (0, 2, 1)).reshape(-1)
    tabf = jnp.transpose(tables.astype(jnp.float32),
                         (0, 2, 1)).reshape(-1)
    resf = jnp.asarray(_resolutions_f32())
    feats = _sc_encode(xt, tabf, resf).reshape(_L * _F, _B)
    out3b = _tc_mlp(
        feats,
        W1.astype(jnp.float32), b1.astype(jnp.float32).reshape(64, 1),
        W2.astype(jnp.float32), b2.astype(jnp.float32).reshape(64, 1),
        W3.astype(jnp.float32), b3.astype(jnp.float32).reshape(3, 1),
    )
    return out3b.T.astype(W1.dtype)


# speedup vs baseline: 1.2514x; 1.0286x over previous
import functools

import numpy as np
import jax
import jax.numpy as jnp
from jax import lax
from jax.experimental import pallas as pl
from jax.experimental.pallas import tpu as pltpu
from jax.experimental.pallas import tpu_sc as plsc

_B = 131072
_L = 16
_T = 14
_F = 2
_TBL = (1 << _T) * _F
_MASK = np.int32((1 << _T) - 1)
_P2 = np.int32(np.int64(2654435761) - (np.int64(1) << np.int64(32)))

_NC = 2
_NS = 16
_LANES = 16
_HALF = _B // _NC
_CHUNK = 8192
_NCHUNK = _HALF // _CHUNK
_NVEC = _CHUNK // _LANES

_CB = 16384
_Z = np.int32(0)


def _resolutions_f32() -> np.ndarray:
    b_geo = np.exp((np.log(512.0) - np.log(16.0)) / (_L - 1))
    return np.array([np.floor(16.0 * b_geo ** i) for i in range(_L)],
                    dtype=np.float32)


def _encode_body(xt_hbm, tab_hbm, res_hbm, out_hbm,
                 tab0_v, tab1_v, res_v, xin_a, xin_b, oc_a, oc_b,
                 sem_tab, sem_in, sem_out):
    c = lax.axis_index("c")
    s = lax.axis_index("s")
    base = c * np.int32(_HALF)
    obase = (s >> np.int32(2)) * np.int32(1 << 20) + ((s & np.int32(3))
                                                      << np.int32(8))

    def in_off(k):
        off = pl.multiple_of(base + np.int32(k * _CHUNK), _CHUNK)
        return pl.multiple_of(off + off, 2 * _CHUNK)

    xbufs = [xin_a, xin_b]
    obufs = [oc_a, oc_b]
    in_cp = [None] * _NCHUNK
    out_cp = [None] * _NCHUNK
    in_cp[0] = pltpu.async_copy(
        xt_hbm.at[pl.ds(in_off(0), 2 * _CHUNK)], xin_a, sem_in)
    half_tbl = np.int32(_TBL // 2)
    tb = pl.multiple_of(s * np.int32(_TBL), _TBL)
    tab_cp0 = pltpu.async_copy(tab_hbm.at[pl.ds(tb, _TBL // 2)], tab0_v, sem_tab)
    tab_cp1 = pltpu.async_copy(
        tab_hbm.at[pl.ds(pl.multiple_of(tb + half_tbl, _TBL // 2), _TBL // 2)],
        tab1_v, sem_tab)
    pltpu.sync_copy(res_hbm, res_v)
    resv = plsc.load_gather(res_v, [jnp.full((_LANES,), s, dtype=jnp.int32)])
    tab_cp0.wait()
    tab_cp1.wait()

    for k in range(_NCHUNK):
        off = pl.multiple_of(base + np.int32(k * _CHUNK), _CHUNK)
        xin_v = xbufs[k % 2]
        oc_v = obufs[k % 2]
        if k + 1 < _NCHUNK:
            in_cp[k + 1] = pltpu.async_copy(
                xt_hbm.at[pl.ds(in_off(k + 1), 2 * _CHUNK)],
                xbufs[(k + 1) % 2], sem_in)
        in_cp[k].wait()
        if k >= 2:
            for cp in out_cp[k - 2]:
                cp.wait()

        @plsc.parallel_loop(np.int32(0), np.int32(_CHUNK),
                            step=np.int32(_LANES), unroll=4)
        def blk_body(st):
            xo = ((st >> np.int32(7)) << np.int32(8)) + (st & np.int32(127))
            if True:
                so = np.int32(0)
                pt = st
                x0 = xin_v[pl.ds(xo, _LANES)]
                x1 = xin_v[pl.ds(xo + np.int32(128), _LANES)]
                fx0 = x0 * resv
                fx1 = x1 * resv
                i0 = fx0.astype(jnp.int32)
                i1 = fx1.astype(jnp.int32)
                f0 = fx0 - i0.astype(jnp.float32)
                f1 = fx1 - i1.astype(jnp.float32)
                one = np.int32(1)
                ha = i1 * _P2
                hb = ha + _P2
                i0p = i0 + one
                e00 = (i0 ^ ha) & _MASK
                e10 = (i0p ^ ha) & _MASK
                e01 = (i0 ^ hb) & _MASK
                e11 = (i0p ^ hb) & _MASK
                v00a = plsc.load_gather(tab0_v, [e00])
                v00b = plsc.load_gather(tab1_v, [e00])
                v10a = plsc.load_gather(tab0_v, [e10])
                v10b = plsc.load_gather(tab1_v, [e10])
                v01a = plsc.load_gather(tab0_v, [e01])
                v01b = plsc.load_gather(tab1_v, [e01])
                v11a = plsc.load_gather(tab0_v, [e11])
                v11b = plsc.load_gather(tab1_v, [e11])
                a0 = v00a + f0 * (v10a - v00a)
                a1 = v01a + f0 * (v11a - v01a)
                b0 = v00b + f0 * (v10b - v00b)
                b1 = v01b + f0 * (v11b - v01b)
                oc_v[pl.ds(xo, _LANES)] = a0 + f1 * (a1 - a0)
                oc_v[pl.ds(xo + np.int32(128), _LANES)] = b0 + f1 * (b1 - b0)

        c0 = obase + ((off >> np.int32(7)) << np.int32(10))
        cps = []
        for j in range(_CHUNK // 128):
            dst = pl.multiple_of(c0 + np.int32(j << 10), 256)
            cps.append(pltpu.async_copy(
                oc_v.at[pl.ds(np.int32(j * 256), 256)],
                out_hbm.at[pl.ds(dst, 256)], sem_out))
        out_cp[k] = cps

    for k in range(max(0, _NCHUNK - 2), _NCHUNK):
        for cp in out_cp[k]:
            cp.wait()


def _sc_encode(xt, tabf, resf):
    mesh = plsc.VectorSubcoreMesh(core_axis_name="c", subcore_axis_name="s",
                                  num_cores=_NC, num_subcores=_NS)
    return pl.kernel(
        _encode_body,
        out_type=jax.ShapeDtypeStruct((_L * _F * _B,), jnp.float32),
        mesh=mesh,
        compiler_params=pltpu.CompilerParams(needs_layout_passes=False),
        scratch_types=[
            pltpu.VMEM((_TBL // 2,), jnp.float32),
            pltpu.VMEM((_TBL // 2,), jnp.float32),
            pltpu.VMEM((_LANES,), jnp.float32),
            pltpu.VMEM((2 * _CHUNK,), jnp.float32),
            pltpu.VMEM((2 * _CHUNK,), jnp.float32),
            pltpu.VMEM((2 * _CHUNK,), jnp.float32),
            pltpu.VMEM((2 * _CHUNK,), jnp.float32),
            pltpu.SemaphoreType.DMA,
            pltpu.SemaphoreType.DMA,
            pltpu.SemaphoreType.DMA,
        ],
    )(xt, tabf, resf)


def _mlp_body(f_ref, w1_ref, b1_ref, w2_ref, b2_ref, w3_ref, b3_ref, o_ref):
    fb = f_ref[...]
    h1 = lax.dot_general(w1_ref[...], fb, (((1,), (0,)), ((), ())),
                         preferred_element_type=jnp.float32) + b1_ref[...]
    h1 = jnp.maximum(h1, 0.0)
    h2 = lax.dot_general(w2_ref[...], h1, (((1,), (0,)), ((), ())),
                         preferred_element_type=jnp.float32) + b2_ref[...]
    h2 = jnp.maximum(h2, 0.0)
    o_ref[...] = lax.dot_general(w3_ref[...], h2, (((1,), (0,)), ((), ())),
                                 preferred_element_type=jnp.float32) + b3_ref[...]


def _tc_mlp(feats, w1, b1, w2, b2, w3, b3):
    grid = _B // _CB
    return pl.pallas_call(
        _mlp_body,
        grid=(grid,),
        in_specs=[
            pl.BlockSpec((_L * _F, _CB), lambda j: (_Z, j)),
            pl.BlockSpec((64, _L * _F), lambda j: (_Z, _Z)),
            pl.BlockSpec((64, 1), lambda j: (_Z, _Z)),
            pl.BlockSpec((64, 64), lambda j: (_Z, _Z)),
            pl.BlockSpec((64, 1), lambda j: (_Z, _Z)),
            pl.BlockSpec((3, 64), lambda j: (_Z, _Z)),
            pl.BlockSpec((3, 1), lambda j: (_Z, _Z)),
        ],
        out_specs=pl.BlockSpec((3, _CB), lambda j: (_Z, j)),
        out_shape=jax.ShapeDtypeStruct((3, _B), jnp.float32),
    )(feats, w1, b1, w2, b2, w3, b3)


def kernel(x, tables, W1, b1, W2, b2, W3, b3):
    xt = jnp.transpose(x.astype(jnp.float32).reshape(_B // 128, 128, 2),
                       (0, 2, 1)).reshape(-1)
    tabf = jnp.transpose(tables.astype(jnp.float32),
                         (0, 2, 1)).reshape(-1)
    resf = jnp.asarray(_resolutions_f32())
    feats = jnp.transpose(
        _sc_encode(xt, tabf, resf).reshape(4, _B // 128, 8, 128),
        (0, 2, 1, 3)).reshape(_L * _F, _B)
    out3b = _tc_mlp(
        feats,
        W1.astype(jnp.float32), b1.astype(jnp.float32).reshape(64, 1),
        W2.astype(jnp.float32), b2.astype(jnp.float32).reshape(64, 1),
        W3.astype(jnp.float32), b3.astype(jnp.float32).reshape(3, 1),
    )
    return out3b.T.astype(W1.dtype)
